# baseline (speedup 1.0000x reference)
import functools

import jax
import jax.numpy as jnp
from jax import lax
from jax.experimental import pallas as pl
from jax.experimental.pallas import tpu as pltpu
from jax.experimental.pallas import tpu_sc as plsc

_CHUNK = 128
_PACK = 4
_NLAYER = 16


_BW = 4096


def _tc_transpose(embT, n_nodes, node_dim):
    g4 = 4 * _BW
    nblk = -(-n_nodes // g4)
    lane_max = -(-n_nodes // _BW) - 1

    def body(x0_ref, x1_ref, x2_ref, x3_ref, out_ref):
        parts = [
            jnp.transpose(r[...]) for r in (x0_ref, x1_ref, x2_ref, x3_ref)
        ]
        out_ref[...] = jnp.concatenate(parts, axis=1)

    def make_map(k):
        return lambda i: (0, jnp.minimum(4 * i + k, lane_max))

    return pl.pallas_call(
        body,
        grid=(nblk,),
        in_specs=[
            pl.BlockSpec((node_dim, _BW), make_map(k)) for k in range(4)
        ],
        out_specs=pl.BlockSpec((_BW, 4 * node_dim), lambda i: (i, 0)),
        out_shape=jax.ShapeDtypeStruct((nblk * _BW, 4 * node_dim), jnp.float32),
    )(embT, embT, embT, embT)


def _remap_idx(u):
    g4 = 4 * _BW
    b = u // g4
    s = u - b * g4
    k = s // _BW
    t = s - k * _BW
    return (b * _BW + t) * 4 + k


def _sc_gather(table, idx2d, total_rows, node_dim):
    info = plsc.get_sparse_core_info()
    nc, ns = info.num_cores, info.num_subcores
    nw = nc * ns
    n_chunks = idx2d.shape[0]
    per_w = n_chunks // nw

    grp = next(g for g in (8, 6, 4, 3, 2, 1) if per_w % (2 * g) == 0)
    n_grp_pairs = per_w // (2 * grp)

    @functools.partial(
        pl.kernel,
        mesh=plsc.VectorSubcoreMesh(core_axis_name="c", subcore_axis_name="s"),
        out_type=jax.ShapeDtypeStruct((total_rows, node_dim), jnp.float32),
        scratch_types=[
            pltpu.VMEM((per_w, _CHUNK), jnp.int32),
            pltpu.VMEM((grp * _CHUNK, node_dim), jnp.float32),
            pltpu.VMEM((grp * _CHUNK, node_dim), jnp.float32),
            pltpu.SemaphoreType.DMA,
            pltpu.SemaphoreType.DMA,
            pltpu.SemaphoreType.DMA,
        ],
        compiler_params=pltpu.CompilerParams(use_tc_tiling_on_sc=False),
    )
    def gather_kernel(idx_hbm, table_hbm, out_hbm, idx_v, ring0, ring1, gsem, osem0, osem1):
        wid = lax.axis_index("s") * nc + lax.axis_index("c")
        pltpu.sync_copy(idx_hbm.at[pl.ds(wid * per_w, per_w)], idx_v)
        rings = (ring0, ring1)
        osems = (osem0, osem1)

        def out_slice(g):
            return out_hbm.at[pl.ds((wid * per_w + g * grp) * _CHUNK, grp * _CHUNK)]

        def run_group(g, ring, osem, drain_prev):
            @pl.when(drain_prev)
            def _():
                pltpu.make_async_copy(ring, out_slice(g), osem).wait()

            gathers = [
                pltpu.async_copy(
                    table_hbm.at[idx_v.at[g * grp + b]],
                    ring.at[pl.ds(b * _CHUNK, _CHUNK)],
                    gsem,
                )
                for b in range(grp)
            ]
            for c in gathers:
                c.wait()
            pltpu.async_copy(ring, out_slice(g), osem)

        def body(p, carry):
            run_group(2 * p, rings[0], osems[0], p >= 1)
            run_group(2 * p + 1, rings[1], osems[1], p >= 1)
            return carry

        lax.fori_loop(0, n_grp_pairs, body, 0)
        g_last = 2 * n_grp_pairs - 1
        pltpu.make_async_copy(rings[0], out_slice(g_last - 1), osems[0]).wait()
        pltpu.make_async_copy(rings[1], out_slice(g_last), osems[1]).wait()

    return gather_kernel(idx2d, table)


def _tc_loss(rows_p, tl3, lb3, W, L_pad, n_blocks, m, node_dim):
    lanes = _PACK * node_dim

    def body(ei_ref, ej_ref, tl_ref, lb_ref, w_ref, l_ref, out_ref):
        i = pl.program_id(0)
        f32 = jnp.float32

        wt = w_ref[...]
        wcol = jnp.concatenate([wt] * _PACK, axis=0)
        wfull = jnp.concatenate([wcol] * _PACK, axis=1)
        rblk = lax.broadcasted_iota(jnp.int32, (lanes, lanes), 0) // node_dim
        cblk = lax.broadcasted_iota(jnp.int32, (lanes, lanes), 1) // node_dim
        wbig = wfull * (rblk == cblk).astype(f32)

        lp = l_ref[...]
        lcol = jnp.concatenate([lp] * _PACK, axis=0)
        lrow = lax.broadcasted_iota(jnp.int32, (_PACK * _NLAYER, 1), 0) // _NLAYER
        lbig = jnp.concatenate(
            [lcol * (lrow == k).astype(f32) for k in range(_PACK)], axis=1
        )

        tl_f = tl_ref[0].astype(f32)
        rsel = (
            lax.broadcasted_iota(jnp.int32, (_PACK, _PACK * _NLAYER), 1) // _NLAYER
            == lax.broadcasted_iota(jnp.int32, (_PACK, _PACK * _NLAYER), 0)
        ).astype(f32)
        tl_rep = jnp.dot(tl_f, rsel, preferred_element_type=f32)
        cmod = (
            lax.broadcasted_iota(jnp.int32, (m, _PACK * _NLAYER), 1) % _NLAYER
        ).astype(f32)
        onehot = (jnp.abs(tl_rep - cmod) < 0.5).astype(f32)
        l_packed = jnp.dot(onehot, lbig, preferred_element_type=f32)

        ai = jnp.dot(ei_ref[...], wbig, preferred_element_type=f32)
        aj = jnp.dot(ej_ref[...], wbig, preferred_element_type=f32)
        ri = ai + l_packed
        rj = aj + l_packed
        s = ri * rj

        seg = (
            lax.broadcasted_iota(jnp.int32, (lanes, _PACK), 0) // node_dim
            == lax.broadcasted_iota(jnp.int32, (lanes, _PACK), 1)
        ).astype(f32)
        ip = jnp.dot(s, seg, preferred_element_type=f32)

        x = lb_ref[0] * ip
        ls = jnp.minimum(x, 0.0) - jnp.log1p(jnp.exp(-jnp.abs(x)))
        part = -jnp.sum(ls, keepdims=True)[:1, :1]

        @pl.when(i == 0)
        def _():
            out_ref[...] = jnp.zeros_like(out_ref)

        out_ref[...] += part

    out = pl.pallas_call(
        body,
        grid=(n_blocks,),
        in_specs=[
            pl.BlockSpec((m, lanes), lambda i: (i, 0)),
            pl.BlockSpec((m, lanes), lambda i: (i + n_blocks, 0)),
            pl.BlockSpec((1, m, _PACK), lambda i: (i, 0, 0)),
            pl.BlockSpec((1, m, _PACK), lambda i: (i, 0, 0)),
            pl.BlockSpec(W.shape, lambda i: (0, 0)),
            pl.BlockSpec((_NLAYER, node_dim), lambda i: (0, 0)),
        ],
        out_specs=pl.BlockSpec((1, 1), lambda i: (0, 0)),
        out_shape=jax.ShapeDtypeStruct((1, 1), jnp.float32),
    )(rows_p, rows_p, tl3, lb3, W, L_pad)
    return out[0, 0]


def kernel(u_i, u_j, this_layer, label, embedding, L_embedding, W):
    b = u_i.shape[0]
    n_nodes, node_dim = embedding.shape
    t2 = _tc_transpose(embedding.T, n_nodes, node_dim)
    lin_table = t2.reshape(-1, node_dim)
    rho_i = _remap_idx(u_i)
    rho_j = _remap_idx(u_j)

    lanes = _PACK * node_dim
    blk = 8192
    L_pad = jnp.zeros((_NLAYER, L_embedding.shape[1]), jnp.float32)
    L_pad = L_pad.at[: L_embedding.shape[0]].set(L_embedding)

    h = b // 2
    loss = jnp.zeros((), jnp.float32)
    for lo in (0, h):
        idx2d = jnp.concatenate(
            [lax.dynamic_slice_in_dim(rho_i, lo, h), lax.dynamic_slice_in_dim(rho_j, lo, h)]
        ).reshape(-1, _CHUNK)
        rows = _sc_gather(lin_table, idx2d, 2 * h, node_dim)
        nb = h // blk
        m = blk // _PACK
        rows_p = rows.reshape(2 * h // _PACK, lanes)
        tl3 = lax.dynamic_slice_in_dim(this_layer, lo, h).reshape(nb, m, _PACK)
        lb3 = lax.dynamic_slice_in_dim(label, lo, h).reshape(nb, m, _PACK)
        loss = loss + _tc_loss(rows_p, tl3, lb3, W, L_pad, nb, m, node_dim)
    return loss


# speedup vs baseline: 2.0519x; 2.0519x over previous
import functools

import jax
import jax.numpy as jnp
from jax import lax
from jax.experimental import pallas as pl
from jax.experimental.pallas import tpu as pltpu
from jax.experimental.pallas import tpu_sc as plsc

_CHUNK = 128
_PACK = 4
_NLAYER = 16


_BW = 4096


def _tc_transpose(embT, n_nodes, node_dim):
    g4 = 4 * _BW
    nblk = -(-n_nodes // g4)
    lane_max = -(-n_nodes // _BW) - 1

    def body(x0_ref, x1_ref, x2_ref, x3_ref, out_ref):
        xb = jnp.concatenate(
            [x0_ref[...], x1_ref[...], x2_ref[...], x3_ref[...]], axis=0
        )
        n = 4 * node_dim
        eye = (
            lax.broadcasted_iota(jnp.int32, (n, n), 0)
            == lax.broadcasted_iota(jnp.int32, (n, n), 1)
        ).astype(jnp.float32)
        out_ref[...] = lax.dot_general(
            xb, eye, (((0,), (0,)), ((), ())), preferred_element_type=jnp.float32
        )

    def make_map(k):
        return lambda i: (0, jnp.minimum(4 * i + k, lane_max))

    return pl.pallas_call(
        body,
        grid=(nblk,),
        in_specs=[
            pl.BlockSpec((node_dim, _BW), make_map(k)) for k in range(4)
        ],
        out_specs=pl.BlockSpec((_BW, 4 * node_dim), lambda i: (i, 0)),
        out_shape=jax.ShapeDtypeStruct((nblk * _BW, 4 * node_dim), jnp.float32),
    )(embT, embT, embT, embT)


def _remap_idx(u):
    g4 = 4 * _BW
    b = u // g4
    s = u - b * g4
    k = s // _BW
    t = s - k * _BW
    return (b * _BW + t) * 4 + k


def _sc_gather(table, idx2d, total_rows, node_dim):
    info = plsc.get_sparse_core_info()
    nc, ns = info.num_cores, info.num_subcores
    nw = nc * ns
    n_chunks = idx2d.shape[0]
    per_w = n_chunks // nw

    grp = next(g for g in (8, 6, 4, 3, 2, 1) if per_w % (2 * g) == 0)
    n_grp_pairs = per_w // (2 * grp)

    @functools.partial(
        pl.kernel,
        mesh=plsc.VectorSubcoreMesh(core_axis_name="c", subcore_axis_name="s"),
        out_type=jax.ShapeDtypeStruct((total_rows, node_dim), jnp.float32),
        scratch_types=[
            pltpu.VMEM((per_w, _CHUNK), jnp.int32),
            pltpu.VMEM((grp * _CHUNK, node_dim), jnp.float32),
            pltpu.VMEM((grp * _CHUNK, node_dim), jnp.float32),
            pltpu.SemaphoreType.DMA,
            pltpu.SemaphoreType.DMA,
            pltpu.SemaphoreType.DMA,
        ],
        compiler_params=pltpu.CompilerParams(use_tc_tiling_on_sc=False),
    )
    def gather_kernel(idx_hbm, table_hbm, out_hbm, idx_v, ring0, ring1, gsem, osem0, osem1):
        wid = lax.axis_index("s") * nc + lax.axis_index("c")
        pltpu.sync_copy(idx_hbm.at[pl.ds(wid * per_w, per_w)], idx_v)
        rings = (ring0, ring1)
        osems = (osem0, osem1)

        def out_slice(g):
            return out_hbm.at[pl.ds((wid * per_w + g * grp) * _CHUNK, grp * _CHUNK)]

        def run_group(g, ring, osem, drain_prev):
            @pl.when(drain_prev)
            def _():
                pltpu.make_async_copy(ring, out_slice(g), osem).wait()

            gathers = [
                pltpu.async_copy(
                    table_hbm.at[idx_v.at[g * grp + b]],
                    ring.at[pl.ds(b * _CHUNK, _CHUNK)],
                    gsem,
                )
                for b in range(grp)
            ]
            for c in gathers:
                c.wait()
            pltpu.async_copy(ring, out_slice(g), osem)

        def body(p, carry):
            run_group(2 * p, rings[0], osems[0], p >= 1)
            run_group(2 * p + 1, rings[1], osems[1], p >= 1)
            return carry

        lax.fori_loop(0, n_grp_pairs, body, 0)
        g_last = 2 * n_grp_pairs - 1
        pltpu.make_async_copy(rings[0], out_slice(g_last - 1), osems[0]).wait()
        pltpu.make_async_copy(rings[1], out_slice(g_last), osems[1]).wait()

    return gather_kernel(idx2d, table)


def _tc_loss(rows_p, tl3, lb3, W, L_pad, n_blocks, m, node_dim):
    lanes = _PACK * node_dim

    def body(ei_ref, ej_ref, tl_ref, lb_ref, w_ref, l_ref, out_ref):
        i = pl.program_id(0)
        f32 = jnp.float32

        wt = w_ref[...]
        wcol = jnp.concatenate([wt] * _PACK, axis=0)
        wfull = jnp.concatenate([wcol] * _PACK, axis=1)
        rblk = lax.broadcasted_iota(jnp.int32, (lanes, lanes), 0) // node_dim
        cblk = lax.broadcasted_iota(jnp.int32, (lanes, lanes), 1) // node_dim
        wbig = wfull * (rblk == cblk).astype(f32)

        lp = l_ref[...]
        lcol = jnp.concatenate([lp] * _PACK, axis=0)
        lrow = lax.broadcasted_iota(jnp.int32, (_PACK * _NLAYER, 1), 0) // _NLAYER
        lbig = jnp.concatenate(
            [lcol * (lrow == k).astype(f32) for k in range(_PACK)], axis=1
        )

        tl_f = tl_ref[0].astype(f32)
        rsel = (
            lax.broadcasted_iota(jnp.int32, (_PACK, _PACK * _NLAYER), 1) // _NLAYER
            == lax.broadcasted_iota(jnp.int32, (_PACK, _PACK * _NLAYER), 0)
        ).astype(f32)
        tl_rep = jnp.dot(tl_f, rsel, preferred_element_type=f32)
        cmod = (
            lax.broadcasted_iota(jnp.int32, (m, _PACK * _NLAYER), 1) % _NLAYER
        ).astype(f32)
        onehot = (jnp.abs(tl_rep - cmod) < 0.5).astype(f32)
        l_packed = jnp.dot(onehot, lbig, preferred_element_type=f32)

        ai = jnp.dot(ei_ref[...], wbig, preferred_element_type=f32)
        aj = jnp.dot(ej_ref[...], wbig, preferred_element_type=f32)
        ri = ai + l_packed
        rj = aj + l_packed
        s = ri * rj

        seg = (
            lax.broadcasted_iota(jnp.int32, (lanes, _PACK), 0) // node_dim
            == lax.broadcasted_iota(jnp.int32, (lanes, _PACK), 1)
        ).astype(f32)
        ip = jnp.dot(s, seg, preferred_element_type=f32)

        x = lb_ref[0] * ip
        ls = jnp.minimum(x, 0.0) - jnp.log1p(jnp.exp(-jnp.abs(x)))
        part = -jnp.sum(ls, keepdims=True)[:1, :1]

        @pl.when(i == 0)
        def _():
            out_ref[...] = jnp.zeros_like(out_ref)

        out_ref[...] += part

    out = pl.pallas_call(
        body,
        grid=(n_blocks,),
        in_specs=[
            pl.BlockSpec((m, lanes), lambda i: (i, 0)),
            pl.BlockSpec((m, lanes), lambda i: (i + n_blocks, 0)),
            pl.BlockSpec((1, m, _PACK), lambda i: (i, 0, 0)),
            pl.BlockSpec((1, m, _PACK), lambda i: (i, 0, 0)),
            pl.BlockSpec(W.shape, lambda i: (0, 0)),
            pl.BlockSpec((_NLAYER, node_dim), lambda i: (0, 0)),
        ],
        out_specs=pl.BlockSpec((1, 1), lambda i: (0, 0)),
        out_shape=jax.ShapeDtypeStruct((1, 1), jnp.float32),
    )(rows_p, rows_p, tl3, lb3, W, L_pad)
    return out[0, 0]


def kernel(u_i, u_j, this_layer, label, embedding, L_embedding, W):
    b = u_i.shape[0]
    n_nodes, node_dim = embedding.shape
    t2 = _tc_transpose(embedding.T, n_nodes, node_dim)
    lin_table = t2.reshape(-1, node_dim)
    rho_i = _remap_idx(u_i)
    rho_j = _remap_idx(u_j)

    lanes = _PACK * node_dim
    blk = 8192
    L_pad = jnp.zeros((_NLAYER, L_embedding.shape[1]), jnp.float32)
    L_pad = L_pad.at[: L_embedding.shape[0]].set(L_embedding)

    idx2d = jnp.concatenate([rho_i, rho_j]).reshape(-1, _CHUNK)
    rows = _sc_gather(lin_table, idx2d, 2 * b, node_dim)
    nb = b // blk
    m = blk // _PACK
    rows_p = rows.reshape(2 * b // _PACK, lanes)
    tl3 = this_layer.reshape(nb, m, _PACK)
    lb3 = label.reshape(nb, m, _PACK)
    return _tc_loss(rows_p, tl3, lb3, W, L_pad, nb, m, node_dim)
